# baseline (speedup 1.0000x reference)
import functools

import jax
import jax.numpy as jnp
from jax import lax
from jax.experimental import pallas as pl
from jax.experimental.pallas import tpu as pltpu
from jax.experimental.pallas import tpu_sc as plsc

_NC = 2
_NS = 16
_LANES = 16
_NW = _NC * _NS


def kernel(x, layer_id, tables):
    n_layers, vocab, d = tables.shape
    b, s = x.shape
    n = b * s
    b_per_w = n // _NW

    flat_tables = tables.reshape(n_layers * vocab, d)
    idx = x.reshape(n)
    off = jnp.full((_LANES,), jnp.int32(layer_id) * vocab, dtype=jnp.int32)

    mesh = plsc.VectorSubcoreMesh(core_axis_name="c", subcore_axis_name="s")

    n_chunks = 4
    chunk = b_per_w // n_chunks

    @functools.partial(
        pl.kernel,
        mesh=mesh,
        out_type=jax.ShapeDtypeStruct((n, d), tables.dtype),
        scratch_types=[
            pltpu.VMEM((b_per_w,), jnp.int32),
            pltpu.VMEM((_LANES,), jnp.int32),
            pltpu.VMEM((chunk, d), jnp.float32),
            pltpu.VMEM((chunk, d), jnp.float32),
            pltpu.SemaphoreType.DMA,
            pltpu.SemaphoreType.DMA,
            pltpu.SemaphoreType.DMA,
            pltpu.SemaphoreType.DMA,
        ],
    )
    def gather_kernel(table_hbm, idx_hbm, off_hbm, out_hbm,
                      idx_v, off_v, rows_a, rows_b,
                      gsem_a, gsem_b, ssem_a, ssem_b):
        wid = lax.axis_index("s") * _NC + lax.axis_index("c")
        base = wid * b_per_w
        pltpu.sync_copy(idx_hbm.at[pl.ds(base, b_per_w)], idx_v)
        pltpu.sync_copy(off_hbm, off_v)
        off_reg = off_v[...]

        @pl.loop(0, b_per_w, step=_LANES)
        def _(i):
            slc = pl.ds(i, _LANES)
            idx_v.at[slc][...] = idx_v.at[slc][...] + off_reg

        bufs = (rows_a, rows_b)
        gsems = (gsem_a, gsem_b)
        ssems = (ssem_a, ssem_b)

        def start_gather(k):
            p = k % 2
            return pltpu.async_copy(
                table_hbm.at[idx_v.at[pl.ds(k * chunk, chunk)]],
                bufs[p], gsems[p])

        def start_store(k):
            p = k % 2
            return pltpu.async_copy(
                bufs[p], out_hbm.at[pl.ds(base + k * chunk, chunk)], ssems[p])

        gathers = [None, None]
        stores = [None, None]
        for k in range(n_chunks):
            p = k % 2
            if k >= 2:
                stores[p].wait()
            gathers[p] = start_gather(k)
            if k >= 1:
                gathers[1 - p].wait()
                stores[1 - p] = start_store(k - 1)
        last = (n_chunks - 1) % 2
        gathers[last].wait()
        stores[last] = start_store(n_chunks - 1)
        stores[1 - last].wait()
        stores[last].wait()

    out = gather_kernel(flat_tables, idx, off)
    return out.reshape(b, s, d)


# speedup vs baseline: 1.0442x; 1.0442x over previous
import functools

import jax
import jax.numpy as jnp
from jax import lax
from jax.experimental import pallas as pl
from jax.experimental.pallas import tpu as pltpu
from jax.experimental.pallas import tpu_sc as plsc

_NC = 2
_NS = 16
_LANES = 16
_NW = _NC * _NS


def kernel(x, layer_id, tables):
    n_layers, vocab, d = tables.shape
    b, s = x.shape
    n = b * s
    b_per_w = n // _NW
    sub_per_row = s // b_per_w
    flat_tables = tables.reshape(n_layers * vocab, d)
    off = jnp.full((_LANES,), jnp.int32(layer_id) * vocab, dtype=jnp.int32)

    mesh = plsc.VectorSubcoreMesh(core_axis_name="c", subcore_axis_name="s")

    @functools.partial(
        pl.kernel,
        mesh=mesh,
        out_type=jax.ShapeDtypeStruct((n, d), tables.dtype),
        scratch_types=[
            pltpu.VMEM((b_per_w,), jnp.int32),
            pltpu.VMEM((_LANES,), jnp.int32),
            pltpu.VMEM((b_per_w, d), jnp.float32),
            pltpu.SemaphoreType.DMA,
        ],
    )
    def gather_kernel(table_hbm, x_hbm, off_hbm, out_hbm,
                      idx_v, off_v, rows_v, sem):
        wid = lax.axis_index("s") * _NC + lax.axis_index("c")
        row = wid // sub_per_row
        col = (wid % sub_per_row) * b_per_w
        pltpu.sync_copy(x_hbm.at[row].at[pl.ds(col, b_per_w)], idx_v)
        pltpu.sync_copy(off_hbm, off_v)
        off_reg = off_v[...]

        @pl.loop(0, b_per_w, step=_LANES)
        def _(i):
            slc = pl.ds(i, _LANES)
            idx_v.at[slc][...] = idx_v.at[slc][...] + off_reg

        pltpu.async_copy(table_hbm.at[idx_v], rows_v, sem).wait()
        pltpu.sync_copy(rows_v, out_hbm.at[pl.ds(wid * b_per_w, b_per_w)])

    out = gather_kernel(flat_tables, x, off)
    return out.reshape(b, s, d)
